# baseline (speedup 1.0000x reference)
import functools

import jax
import jax.numpy as jnp
from jax import lax
from jax.experimental import pallas as pl
from jax.experimental.pallas import tpu as pltpu
from jax.experimental.pallas import tpu_sc as plsc

B = 16384
K = 20
D = 32
DP = D // 2
NW = 32
ROWS_W = B // NW
C = 64
NCHUNK = ROWS_W // C
NIDX_ROWS = C * K // 128
G = C // 16


def _unpack_cols(packed):
    pair = plsc.bitcast(packed, jnp.bfloat16)
    return plsc.unpack(pair, format=plsc.PackFormat.INTERLEAVED)


def _body(uidx_hbm, pidx_hbm, nidx_hbm, utab, itab, pos_out, neg_out,
          uidx_v, pidx_v, nidx_v, urows_v, prows_v, nrows_v, pout_v, nout_v,
          sem):
    cid = lax.axis_index("c")
    sid = lax.axis_index("s")
    wid = sid * 2 + cid
    l16 = lax.iota(jnp.int32, 16)
    cds = [jnp.full((16,), d, jnp.int32) for d in range(DP)]

    def chunk_body(c, carry):
        base = wid * ROWS_W + c * C
        pltpu.sync_copy(uidx_hbm.at[pl.ds(base, C)], uidx_v)
        pltpu.sync_copy(pidx_hbm.at[pl.ds(base, C)], pidx_v)
        for j in range(NIDX_ROWS):
            pltpu.sync_copy(nidx_hbm.at[pl.ds(base * K + j * 128, 128)],
                            nidx_v.at[j])
        hs = [pltpu.async_copy(utab.at[uidx_v], urows_v, sem),
              pltpu.async_copy(itab.at[pidx_v], prows_v, sem)]
        for j in range(NIDX_ROWS):
            hs.append(pltpu.async_copy(itab.at[nidx_v.at[j]],
                                       nrows_v.at[pl.ds(j * 128, 128)], sem))
        for h in hs:
            h.wait()

        def group_body(g, gcarry):
            rowb = g * 16 + l16
            ucols = []
            for d in range(DP):
                ucols.extend(_unpack_cols(
                    plsc.load_gather(urows_v, [rowb, cds[d]])))
            pe0, po0 = _unpack_cols(plsc.load_gather(prows_v, [rowb, cds[0]]))
            pe1, po1 = _unpack_cols(plsc.load_gather(prows_v, [rowb, cds[1]]))
            accp0 = ucols[0] * pe0 + ucols[1] * po0
            accp1 = ucols[2] * pe1 + ucols[3] * po1
            for d in range(2, DP, 2):
                pe0, po0 = _unpack_cols(
                    plsc.load_gather(prows_v, [rowb, cds[d]]))
                pe1, po1 = _unpack_cols(
                    plsc.load_gather(prows_v, [rowb, cds[d + 1]]))
                accp0 = accp0 + (ucols[2 * d] * pe0 + ucols[2 * d + 1] * po0)
                accp1 = accp1 + (ucols[2 * d + 2] * pe1
                                 + ucols[2 * d + 3] * po1)
            pout_v[pl.ds(g * 16, 16)] = accp0 + accp1

            def k_body(k, kcarry):
                rowbk = rowb * K + k
                ne0, no0 = _unpack_cols(
                    plsc.load_gather(nrows_v, [rowbk, cds[0]]))
                ne1, no1 = _unpack_cols(
                    plsc.load_gather(nrows_v, [rowbk, cds[1]]))
                accn0 = ucols[0] * ne0 + ucols[1] * no0
                accn1 = ucols[2] * ne1 + ucols[3] * no1
                for d in range(2, DP, 2):
                    ne0, no0 = _unpack_cols(
                        plsc.load_gather(nrows_v, [rowbk, cds[d]]))
                    ne1, no1 = _unpack_cols(
                        plsc.load_gather(nrows_v, [rowbk, cds[d + 1]]))
                    accn0 = accn0 + (ucols[2 * d] * ne0
                                     + ucols[2 * d + 1] * no0)
                    accn1 = accn1 + (ucols[2 * d + 2] * ne1
                                     + ucols[2 * d + 3] * no1)
                plsc.store_scatter(
                    nout_v, [rowb, jnp.full((16,), 0, jnp.int32) + k],
                    accn0 + accn1)
                return kcarry

            lax.fori_loop(0, K, k_body, 0)
            return gcarry

        lax.fori_loop(0, G, group_body, 0)
        pltpu.sync_copy(pout_v, pos_out.at[pl.ds(base, C)])
        pltpu.sync_copy(nout_v, neg_out.at[pl.ds(base, C)])
        return carry

    lax.fori_loop(0, NCHUNK, chunk_body, 0)


@jax.jit
def _sc_call(user, pos_item, neg_flat, utab_i32, itab_i32):
    mesh = plsc.VectorSubcoreMesh(core_axis_name="c", subcore_axis_name="s")
    kfn = functools.partial(
        pl.kernel,
        out_type=[jax.ShapeDtypeStruct((B,), jnp.float32),
                  jax.ShapeDtypeStruct((B, K), jnp.float32)],
        mesh=mesh,
        scratch_types=[
            pltpu.VMEM((C,), jnp.int32),
            pltpu.VMEM((C,), jnp.int32),
            pltpu.VMEM((NIDX_ROWS, 128), jnp.int32),
            pltpu.VMEM((C, DP), jnp.int32),
            pltpu.VMEM((C, DP), jnp.int32),
            pltpu.VMEM((C * K, DP), jnp.int32),
            pltpu.VMEM((C,), jnp.float32),
            pltpu.VMEM((C, K), jnp.float32),
            pltpu.SemaphoreType.DMA,
        ],
        compiler_params=pltpu.CompilerParams(needs_layout_passes=False,
                                             use_tc_tiling_on_sc=False),
    )(_body)
    return kfn(user, pos_item, neg_flat, utab_i32, itab_i32)


def kernel(user, pos_item, neg_items, user_table, item_table):
    user = user.astype(jnp.int32)
    pos_item = pos_item.astype(jnp.int32)
    neg_flat = neg_items.astype(jnp.int32).reshape(B * K)
    utab_i32 = lax.bitcast_convert_type(
        user_table.astype(jnp.bfloat16).reshape(1000000, DP, 2), jnp.int32)
    itab_i32 = lax.bitcast_convert_type(
        item_table.astype(jnp.bfloat16).reshape(1000000, DP, 2), jnp.int32)
    pos_s, neg_s = _sc_call(user, pos_item, neg_flat, utab_i32, itab_i32)
    return (pos_s, neg_s)


# speedup vs baseline: 1.9352x; 1.9352x over previous
import functools

import jax
import jax.numpy as jnp
from jax import lax
from jax.experimental import pallas as pl
from jax.experimental.pallas import tpu as pltpu
from jax.experimental.pallas import tpu_sc as plsc

B = 16384
K = 20
D = 32
NW = 32
ROWS_W = B // NW
C = 64
NCHUNK = ROWS_W // C
NIDX_ROWS = C * K // 128
G = C // 16


def _body(uidx_hbm, pidx_hbm, nidx_hbm, utab, itab, pos_out, neg_out,
          uidx_v, pidx_v, nidx_v, urows_v, prows_v, nrows_v, pout_v, nout_v,
          sem_a, sem_b):
    cid = lax.axis_index("c")
    sid = lax.axis_index("s")
    wid = sid * 2 + cid
    l16 = lax.iota(jnp.int32, 16)
    cds = [jnp.full((16,), d, jnp.int32) for d in range(D)]
    sems = [sem_a, sem_b]

    def stage_and_fire(c):
        p = c % 2
        base = wid * ROWS_W + c * C
        pltpu.sync_copy(uidx_hbm.at[pl.ds(base, C)], uidx_v.at[p])
        pltpu.sync_copy(pidx_hbm.at[pl.ds(base, C)], pidx_v.at[p])
        for j in range(NIDX_ROWS):
            pltpu.sync_copy(nidx_hbm.at[pl.ds(base * K + j * 128, 128)],
                            nidx_v.at[p, j])
        hs = [pltpu.async_copy(utab.at[uidx_v.at[p]], urows_v.at[p], sems[p]),
              pltpu.async_copy(itab.at[pidx_v.at[p]], prows_v.at[p], sems[p])]
        for j in range(NIDX_ROWS):
            hs.append(pltpu.async_copy(
                itab.at[nidx_v.at[p, j]],
                nrows_v.at[p, pl.ds(j * 128, 128)], sems[p]))
        return hs

    def compute(c):
        p = c % 2
        base = wid * ROWS_W + c * C
        urows = urows_v.at[p]
        prows = prows_v.at[p]
        nrows = nrows_v.at[p]

        def group_body(g, gcarry):
            rowb = g * 16 + l16
            ucols = [plsc.load_gather(urows, [rowb, cds[d]])
                     for d in range(D)]
            accp0 = ucols[0] * plsc.load_gather(prows, [rowb, cds[0]])
            accp1 = ucols[1] * plsc.load_gather(prows, [rowb, cds[1]])
            for d in range(2, D, 2):
                accp0 = accp0 + ucols[d] * plsc.load_gather(
                    prows, [rowb, cds[d]])
                accp1 = accp1 + ucols[d + 1] * plsc.load_gather(
                    prows, [rowb, cds[d + 1]])
            pout_v[pl.ds(g * 16, 16)] = accp0 + accp1

            def k_body(k, kcarry):
                rowbk = rowb * K + k
                accn0 = ucols[0] * plsc.load_gather(nrows, [rowbk, cds[0]])
                accn1 = ucols[1] * plsc.load_gather(nrows, [rowbk, cds[1]])
                for d in range(2, D, 2):
                    accn0 = accn0 + ucols[d] * plsc.load_gather(
                        nrows, [rowbk, cds[d]])
                    accn1 = accn1 + ucols[d + 1] * plsc.load_gather(
                        nrows, [rowbk, cds[d + 1]])
                plsc.store_scatter(
                    nout_v, [rowb, jnp.full((16,), 0, jnp.int32) + k],
                    accn0 + accn1)
                return kcarry

            lax.fori_loop(0, K, k_body, 0)
            return gcarry

        lax.fori_loop(0, G, group_body, 0)
        pltpu.sync_copy(pout_v, pos_out.at[pl.ds(base, C)])
        pltpu.sync_copy(nout_v, neg_out.at[pl.ds(base, C)])

    hs = stage_and_fire(0)
    for c in range(NCHUNK):
        nxt = stage_and_fire(c + 1) if c + 1 < NCHUNK else []
        for h in hs:
            h.wait()
        compute(c)
        hs = nxt


@jax.jit
def _sc_call(user, pos_item, neg_flat, utab, itab):
    mesh = plsc.VectorSubcoreMesh(core_axis_name="c", subcore_axis_name="s")
    kfn = functools.partial(
        pl.kernel,
        out_type=[jax.ShapeDtypeStruct((B,), jnp.float32),
                  jax.ShapeDtypeStruct((B, K), jnp.float32)],
        mesh=mesh,
        scratch_types=[
            pltpu.VMEM((2, C), jnp.int32),
            pltpu.VMEM((2, C), jnp.int32),
            pltpu.VMEM((2, NIDX_ROWS, 128), jnp.int32),
            pltpu.VMEM((2, C, D), jnp.float32),
            pltpu.VMEM((2, C, D), jnp.float32),
            pltpu.VMEM((2, C * K, D), jnp.float32),
            pltpu.VMEM((C,), jnp.float32),
            pltpu.VMEM((C, K), jnp.float32),
            pltpu.SemaphoreType.DMA,
            pltpu.SemaphoreType.DMA,
        ],
        compiler_params=pltpu.CompilerParams(needs_layout_passes=False,
                                             use_tc_tiling_on_sc=False),
    )(_body)
    return kfn(user, pos_item, neg_flat, utab, itab)


def kernel(user, pos_item, neg_items, user_table, item_table):
    user = user.astype(jnp.int32)
    pos_item = pos_item.astype(jnp.int32)
    neg_flat = neg_items.astype(jnp.int32).reshape(B * K)
    pos_s, neg_s = _sc_call(user, pos_item, neg_flat, user_table, item_table)
    return (pos_s, neg_s)
